# baseline (speedup 1.0000x reference)
import jax
import jax.numpy as jnp
from jax.experimental import pallas as pl
from jax.experimental.pallas import tpu as pltpu
from jax.experimental.pallas import tpu_sc as plsc

_EMBEDDING_DIM = 128
_STEPS = 32
_N_FREQS = 16
_N_ATOMS = 256
_TIME = 1024
_BATCH = 2
_NB = 2 * _BATCH
_NEG = float("-inf")

_L = 16
_NCORE = 2
_NSUB = 16
_PER_BATCH = _N_ATOMS * _TIME
_W_PER_B = 8
_CHUNK = _PER_BATCH // _W_PER_B
_NVEC = _CHUNK // _L
_NGRP = 32
_GVEC = _NVEC // _NGRP


def _pos_encode(x, n_freqs=_N_FREQS):
    outs = [x]
    for i in range(n_freqs):
        outs.append(jnp.sin((2.0 ** i) * x))
        outs.append(jnp.cos((2.0 ** i) * x))
    return jnp.concatenate(outs, axis=-1)


def _scal(x):
    return x if x.ndim == 0 else jax.lax.reduce_max(x, (0,))


def _sc_top32_body(a_ref, b_ref, vals_ref, flat_ref, stv_ref, stf_ref,
                   chunk, s1, lv, lf, mv, mf):
    c = jax.lax.axis_index("c")
    s = jax.lax.axis_index("s")
    batch = c * 2 + s // _W_PER_B
    w = s % _W_PER_B
    off = (batch % 2) * _PER_BATCH + w * _CHUNK

    @pl.when(batch < 2)
    def _():
        pltpu.sync_copy(a_ref.at[pl.ds(off, _CHUNK)], chunk)

    @pl.when(batch >= 2)
    def _():
        pltpu.sync_copy(b_ref.at[pl.ds(off, _CHUNK)], chunk)

    iota = jax.lax.broadcasted_iota(jnp.int32, (_L,), 0)
    neg = jnp.float32(_NEG)
    big = jnp.int32(1 << 30)

    def build_g(g, _):
        def inner(j8, m):
            for u in range(8):
                v = plsc.load_gather(
                    chunk, [(g * _GVEC + j8 * 8 + u) * _L + iota])
                m = jnp.maximum(m, v)
            return m
        m = jax.lax.fori_loop(0, _GVEC // 8, inner, jnp.full((_L,), neg))
        plsc.store_scatter(s1, [g * _L + iota], m)
        return 0

    jax.lax.fori_loop(0, _NGRP, build_g, 0)

    def build_m(g, m):
        return jnp.maximum(m, plsc.load_gather(s1, [g * _L + iota]))

    m_run = jax.lax.fori_loop(0, _NGRP, build_m, jnp.full((_L,), neg))

    def extract(i, m_vec):
        m = jax.lax.reduce_max(m_vec, (0,))
        lane = _scal(plsc.all_reduce_ffs(m_vec == m))
        col0 = plsc.load_gather(s1, [lane + _L * iota])
        col1 = plsc.load_gather(s1, [lane + _L * (iota + _L)])
        m0 = col0 == m
        g = jnp.where(jnp.any(m0),
                      _scal(plsc.all_reduce_ffs(m0)),
                      _L + _scal(plsc.all_reduce_ffs(col1 == m)))
        gbase = g * _GVEC
        vsel = big
        for q in range(4):
            idxq = (gbase + q * _L + iota) * _L + lane
            vq = plsc.load_gather(chunk, [idxq])
            cand = jnp.where(vq == m, gbase + q * _L + iota, big)
            vsel = jnp.minimum(vsel, jax.lax.reduce_min(cand, (0,)))
        flat = vsel * _L + lane
        plsc.store_scatter(chunk, [flat + iota * 0], jnp.full((_L,), neg),
                           mask=iota == 0)
        gm = neg
        for q in range(4):
            idxq = (gbase + q * _L + iota) * _L + lane
            vq = plsc.load_gather(chunk, [idxq])
            gm = jnp.maximum(gm, jax.lax.reduce_max(vq, (0,)))
        plsc.store_scatter(s1, [g * _L + lane + iota * 0],
                           jnp.full((_L,), gm), mask=iota == 0)
        col0u = jnp.where(iota == g, jnp.full((_L,), gm), col0)
        col1u = jnp.where(iota + _L == g, jnp.full((_L,), gm), col1)
        cmax = jnp.maximum(jax.lax.reduce_max(col0u, (0,)),
                           jax.lax.reduce_max(col1u, (0,)))
        m_vec = jnp.where(iota == lane, cmax, m_vec)
        plsc.store_scatter(lv, [i + iota * 0], jnp.full((_L,), m),
                           mask=iota == 0)
        plsc.store_scatter(lf, [i + iota * 0],
                           jnp.full((_L,), w * _CHUNK + flat),
                           mask=iota == 0)
        return m_vec

    jax.lax.fori_loop(0, _STEPS, extract, m_run)

    wid = c * _NSUB + s
    pltpu.sync_copy(lv, stv_ref.at[pl.ds(wid * _STEPS, _STEPS)])
    pltpu.sync_copy(lf, stf_ref.at[pl.ds(wid * _STEPS, _STEPS)])
    plsc.subcore_barrier()

    @pl.when(w == 0)
    def _():
        for k in range(_W_PER_B):
            pltpu.sync_copy(stv_ref.at[pl.ds((wid + k) * _STEPS, _STEPS)],
                            mv.at[pl.ds(k * _STEPS, _STEPS)])
            pltpu.sync_copy(stf_ref.at[pl.ds((wid + k) * _STEPS, _STEPS)],
                            mf.at[pl.ds(k * _STEPS, _STEPS)])

        def mstep(i, _):
            vecs = [plsc.load_gather(mv, [v * _L + iota]) for v in range(16)]
            m_vec = vecs[0]
            for v in range(1, 16):
                m_vec = jnp.maximum(m_vec, vecs[v])
            m = jax.lax.reduce_max(m_vec, (0,))
            fsel = big
            for v in range(16):
                fv = plsc.load_gather(mf, [v * _L + iota])
                fsel = jnp.minimum(fsel, jax.lax.reduce_min(
                    jnp.where(vecs[v] == m, fv, big), (0,)))
            for v in range(16):
                idx = v * _L + iota
                fv = plsc.load_gather(mf, [idx])
                kill = (vecs[v] == m) & (fv == fsel)
                plsc.store_scatter(mv, [idx], jnp.full((_L,), neg), mask=kill)
            plsc.store_scatter(lv, [i + iota * 0], jnp.full((_L,), m),
                               mask=iota == 0)
            plsc.store_scatter(lf, [i + iota * 0], jnp.full((_L,), fsel),
                               mask=iota == 0)
            return 0

        jax.lax.fori_loop(0, _STEPS, mstep, 0)
        pltpu.sync_copy(lv, vals_ref.at[pl.ds(batch * _STEPS, _STEPS)])
        pltpu.sync_copy(lf, flat_ref.at[pl.ds(batch * _STEPS, _STEPS)])


def _assemble_body(emb_ref, keys_s_ref, keys_l_ref, out_ref):
    emb = emb_ref[...]
    keys = keys_s_ref[...]
    keys_t = keys_l_ref[...]
    it_k = jax.lax.broadcasted_iota(jnp.int32, (_NB, _STEPS, 1), 1)
    it_kp = jax.lax.broadcasted_iota(jnp.int32, (_NB, 1, _STEPS), 2)
    less = keys_t < keys
    tie = (keys_t == keys) & (it_kp < it_k)
    ranks = jnp.sum((less | tie).astype(jnp.int32), axis=2, keepdims=True)
    perm = (ranks == it_kp).astype(jnp.float32)
    emb_sorted = jax.lax.dot_general(
        perm, emb, (((1,), (1,)), ((0,), (0,))),
        precision=jax.lax.Precision.HIGHEST,
        preferred_element_type=jnp.float32)
    diff = emb_sorted[:_BATCH] - emb_sorted[_BATCH:]
    total = jnp.sum(diff * diff, axis=(0, 1, 2), keepdims=True)
    out_ref[...] = total.reshape(1, 1) / (_BATCH * _STEPS * _EMBEDDING_DIM)


def kernel(a, b, embeddings, ordering_w):
    mesh = plsc.VectorSubcoreMesh(core_axis_name="c", subcore_axis_name="s",
                                  num_cores=_NCORE, num_subcores=_NSUB)
    vals, flat, _, _ = pl.kernel(
        _sc_top32_body,
        out_type=[
            jax.ShapeDtypeStruct((_NB * _STEPS,), jnp.float32),
            jax.ShapeDtypeStruct((_NB * _STEPS,), jnp.int32),
            jax.ShapeDtypeStruct((_NCORE * _NSUB * _STEPS,), jnp.float32),
            jax.ShapeDtypeStruct((_NCORE * _NSUB * _STEPS,), jnp.int32),
        ],
        mesh=mesh,
        compiler_params=pltpu.CompilerParams(needs_layout_passes=False),
        scratch_types=[
            pltpu.VMEM((_CHUNK,), jnp.float32),
            pltpu.VMEM((_NGRP * _L,), jnp.float32),
            pltpu.VMEM((_STEPS,), jnp.float32),
            pltpu.VMEM((_STEPS,), jnp.int32),
            pltpu.VMEM((_W_PER_B * _STEPS,), jnp.float32),
            pltpu.VMEM((_W_PER_B * _STEPS,), jnp.int32),
        ],
    )(a.reshape(-1), b.reshape(-1))

    vals = vals.reshape(_NB, _STEPS)
    flat = flat.reshape(_NB, _STEPS)
    aidx = flat // _TIME
    tidx = flat - aidx * _TIME

    rng = jnp.linspace(0.0, 1.0, _TIME)
    scalar_pos = rng[tidx]
    pos_enc = _pos_encode(scalar_pos[..., None])
    v_enc = _pos_encode(vals[..., None])
    a_emb = embeddings[aidx]
    emb = jnp.concatenate([pos_enc, v_enc, a_emb], axis=-1)
    keys = emb @ ordering_w

    out = pl.pallas_call(
        _assemble_body,
        out_shape=jax.ShapeDtypeStruct((1, 1), jnp.float32),
    )(emb, keys.reshape(_NB, _STEPS, 1), keys.reshape(_NB, 1, _STEPS))
    return out.reshape(())


# speedup vs baseline: 1.0995x; 1.0995x over previous
import jax
import jax.numpy as jnp
from jax.experimental import pallas as pl
from jax.experimental.pallas import tpu as pltpu
from jax.experimental.pallas import tpu_sc as plsc

_EMBEDDING_DIM = 128
_STEPS = 32
_N_FREQS = 16
_N_ATOMS = 256
_TIME = 1024
_BATCH = 2
_NB = 2 * _BATCH
_NEG = float("-inf")

_L = 16
_NCORE = 2
_NSUB = 16
_PER_BATCH = _N_ATOMS * _TIME
_W_PER_B = 8
_CHUNK = _PER_BATCH // _W_PER_B
_NVEC = _CHUNK // _L
_NGRP = 32
_GVEC = _NVEC // _NGRP


def _pos_encode(x, n_freqs=_N_FREQS):
    outs = [x]
    for i in range(n_freqs):
        outs.append(jnp.sin((2.0 ** i) * x))
        outs.append(jnp.cos((2.0 ** i) * x))
    return jnp.concatenate(outs, axis=-1)


def _scal(x):
    return x if x.ndim == 0 else jax.lax.reduce_max(x, (0,))


def _sc_top32_body(a_ref, b_ref, vals_ref, flat_ref, stv_ref, stf_ref,
                   chunk, s1, lv, lf, mv, mf):
    c = jax.lax.axis_index("c")
    s = jax.lax.axis_index("s")
    batch = c * 2 + s // _W_PER_B
    w = s % _W_PER_B
    off = (batch % 2) * _PER_BATCH + w * _CHUNK

    @pl.when(batch < 2)
    def _():
        pltpu.sync_copy(a_ref.at[pl.ds(off, _CHUNK)], chunk)

    @pl.when(batch >= 2)
    def _():
        pltpu.sync_copy(b_ref.at[pl.ds(off, _CHUNK)], chunk)

    iota = jax.lax.broadcasted_iota(jnp.int32, (_L,), 0)
    neg = jnp.float32(_NEG)
    big = jnp.int32(1 << 30)

    def build_g(g, acc):
        def inner(j16, m):
            for u in range(16):
                v = plsc.load_gather(
                    chunk, [(g * _GVEC + j16 * 16 + u) * _L + iota])
                m = jnp.maximum(m, v)
            return m
        m = jax.lax.fori_loop(0, _GVEC // 16, inner, jnp.full((_L,), neg))
        plsc.store_scatter(s1, [g * _L + iota], m)
        return jnp.maximum(acc, m)

    m_run = jax.lax.fori_loop(0, _NGRP, build_g, jnp.full((_L,), neg))

    def extract(i, m_vec):
        m = jax.lax.reduce_max(m_vec, (0,))
        lane = _scal(plsc.all_reduce_ffs(m_vec == m))
        col0 = plsc.load_gather(s1, [lane + _L * iota])
        col1 = plsc.load_gather(s1, [lane + _L * (iota + _L)])
        cg = jnp.minimum(jnp.where(col0 == m, iota, big),
                         jnp.where(col1 == m, iota + _L, big))
        g = jax.lax.reduce_min(cg, (0,))
        gbase = g * _GVEC
        vqs, cand = [], None
        for q in range(4):
            idxq = (gbase + q * _L + iota) * _L + lane
            vq = plsc.load_gather(chunk, [idxq])
            vqs.append(vq)
            cq = jnp.where(vq == m, gbase + q * _L + iota, big)
            cand = cq if cand is None else jnp.minimum(cand, cq)
        vsel = jax.lax.reduce_min(cand, (0,))
        flat = vsel * _L + lane
        plsc.store_scatter(chunk, [flat + iota * 0], jnp.full((_L,), neg),
                           mask=iota == 0)
        gm_vec = None
        for q in range(4):
            vv = jnp.where((gbase + q * _L + iota) == vsel, neg, vqs[q])
            gm_vec = vv if gm_vec is None else jnp.maximum(gm_vec, vv)
        gm = jax.lax.reduce_max(gm_vec, (0,))
        plsc.store_scatter(s1, [g * _L + lane + iota * 0],
                           jnp.full((_L,), gm), mask=iota == 0)
        col0u = jnp.where(iota == g, gm, col0)
        col1u = jnp.where(iota + _L == g, gm, col1)
        cmax = jax.lax.reduce_max(jnp.maximum(col0u, col1u), (0,))
        m_vec = jnp.where(iota == lane, cmax, m_vec)
        plsc.store_scatter(lv, [i + iota * 0], jnp.full((_L,), m),
                           mask=iota == 0)
        plsc.store_scatter(lf, [i + iota * 0],
                           jnp.full((_L,), w * _CHUNK + flat),
                           mask=iota == 0)
        return m_vec

    jax.lax.fori_loop(0, _STEPS, extract, m_run)

    wid = c * _NSUB + s
    pltpu.sync_copy(lv, stv_ref.at[pl.ds(wid * _STEPS, _STEPS)])
    pltpu.sync_copy(lf, stf_ref.at[pl.ds(wid * _STEPS, _STEPS)])
    plsc.subcore_barrier()

    @pl.when(w == 0)
    def _():
        for k in range(_W_PER_B):
            pltpu.sync_copy(stv_ref.at[pl.ds((wid + k) * _STEPS, _STEPS)],
                            mv.at[pl.ds(k * _STEPS, _STEPS)])
            pltpu.sync_copy(stf_ref.at[pl.ds((wid + k) * _STEPS, _STEPS)],
                            mf.at[pl.ds(k * _STEPS, _STEPS)])

        in8 = iota < _W_PER_B
        p0 = jnp.where(in8, iota * _STEPS, 0)
        h0 = jnp.where(in8, plsc.load_gather(mv, [p0]), neg)
        f0 = jnp.where(in8, plsc.load_gather(mf, [p0]), big)

        def mstep(i, carry):
            h, hf, p = carry
            m = jax.lax.reduce_max(h, (0,))
            fsel = jax.lax.reduce_min(jnp.where(h == m, hf, big), (0,))
            win = (h == m) & (hf == fsel)
            p2 = jnp.where(win, p + 1, p)
            idx = jnp.minimum(p2, _W_PER_B * _STEPS - 1)
            nh = plsc.load_gather(mv, [idx])
            nf = plsc.load_gather(mf, [idx])
            done = p2 >= (iota + 1) * _STEPS
            h = jnp.where(win, jnp.where(done, neg, nh), h)
            hf = jnp.where(win, jnp.where(done, big, nf), hf)
            plsc.store_scatter(lv, [i + iota * 0], jnp.full((_L,), m),
                               mask=iota == 0)
            plsc.store_scatter(lf, [i + iota * 0], jnp.full((_L,), fsel),
                               mask=iota == 0)
            return h, hf, p2

        jax.lax.fori_loop(0, _STEPS, mstep, (h0, f0, p0))
        pltpu.sync_copy(lv, vals_ref.at[pl.ds(batch * _STEPS, _STEPS)])
        pltpu.sync_copy(lf, flat_ref.at[pl.ds(batch * _STEPS, _STEPS)])


def _assemble_body(emb_ref, keys_s_ref, keys_l_ref, out_ref):
    emb = emb_ref[...]
    keys = keys_s_ref[...]
    keys_t = keys_l_ref[...]
    it_k = jax.lax.broadcasted_iota(jnp.int32, (_NB, _STEPS, 1), 1)
    it_kp = jax.lax.broadcasted_iota(jnp.int32, (_NB, 1, _STEPS), 2)
    less = keys_t < keys
    tie = (keys_t == keys) & (it_kp < it_k)
    ranks = jnp.sum((less | tie).astype(jnp.int32), axis=2, keepdims=True)
    perm = (ranks == it_kp).astype(jnp.float32)
    emb_sorted = jax.lax.dot_general(
        perm, emb, (((1,), (1,)), ((0,), (0,))),
        precision=jax.lax.Precision.HIGHEST,
        preferred_element_type=jnp.float32)
    diff = emb_sorted[:_BATCH] - emb_sorted[_BATCH:]
    total = jnp.sum(diff * diff, axis=(0, 1, 2), keepdims=True)
    out_ref[...] = total.reshape(1, 1) / (_BATCH * _STEPS * _EMBEDDING_DIM)


def kernel(a, b, embeddings, ordering_w):
    mesh = plsc.VectorSubcoreMesh(core_axis_name="c", subcore_axis_name="s",
                                  num_cores=_NCORE, num_subcores=_NSUB)
    vals, flat, _, _ = pl.kernel(
        _sc_top32_body,
        out_type=[
            jax.ShapeDtypeStruct((_NB * _STEPS,), jnp.float32),
            jax.ShapeDtypeStruct((_NB * _STEPS,), jnp.int32),
            jax.ShapeDtypeStruct((_NCORE * _NSUB * _STEPS,), jnp.float32),
            jax.ShapeDtypeStruct((_NCORE * _NSUB * _STEPS,), jnp.int32),
        ],
        mesh=mesh,
        compiler_params=pltpu.CompilerParams(needs_layout_passes=False),
        scratch_types=[
            pltpu.VMEM((_CHUNK,), jnp.float32),
            pltpu.VMEM((_NGRP * _L,), jnp.float32),
            pltpu.VMEM((_STEPS,), jnp.float32),
            pltpu.VMEM((_STEPS,), jnp.int32),
            pltpu.VMEM((_W_PER_B * _STEPS,), jnp.float32),
            pltpu.VMEM((_W_PER_B * _STEPS,), jnp.int32),
        ],
    )(a.reshape(-1), b.reshape(-1))

    vals = vals.reshape(_NB, _STEPS)
    flat = flat.reshape(_NB, _STEPS)
    aidx = flat // _TIME
    tidx = flat - aidx * _TIME

    rng = jnp.linspace(0.0, 1.0, _TIME)
    scalar_pos = rng[tidx]
    pos_enc = _pos_encode(scalar_pos[..., None])
    v_enc = _pos_encode(vals[..., None])
    a_emb = embeddings[aidx]
    emb = jnp.concatenate([pos_enc, v_enc, a_emb], axis=-1)
    keys = emb @ ordering_w

    out = pl.pallas_call(
        _assemble_body,
        out_shape=jax.ShapeDtypeStruct((1, 1), jnp.float32),
    )(emb, keys.reshape(_NB, _STEPS, 1), keys.reshape(_NB, 1, _STEPS))
    return out.reshape(())
